# baseline (speedup 1.0000x reference)
import functools

import jax
import jax.numpy as jnp
from jax import lax
from jax.experimental import pallas as pl
from jax.experimental.pallas import tpu as pltpu, tpu_sc as plsc

M, P, D = 16, 1024, 768
NC, NS, L = 2, 16, 16
NW = NC * NS
PPW = P // NW
SL = D // L

_mesh = plsc.VectorSubcoreMesh(core_axis_name="c", subcore_axis_name="s")


@functools.partial(
    pl.kernel,
    out_type=jax.ShapeDtypeStruct((M, P, D), jnp.float32),
    mesh=_mesh,
    scratch_types=[
        pltpu.VMEM((PPW, D), jnp.float32),
        pltpu.VMEM((PPW, D), jnp.float32),
        pltpu.VMEM((M, PPW * L), jnp.float32),
        pltpu.VMEM((PPW, D), jnp.float32),
        pltpu.VMEM((PPW, D), jnp.float32),
        pltpu.SemaphoreType.DMA,
        pltpu.SemaphoreType.DMA,
        pltpu.SemaphoreType.DMA,
    ],
)
def _sc_kernel(feat_hbm, pos_hbm, mask_hbm, out_hbm,
               a_v, b_v, mask_v, ob0_v, ob1_v, sem0, sem1, sem_in):
    wid = lax.axis_index("s") * NC + lax.axis_index("c")
    base = wid * PPW
    cp_a = pltpu.async_copy(feat_hbm.at[pl.ds(base, PPW)], a_v, sem0)
    cp_b = pltpu.async_copy(pos_hbm.at[pl.ds(base, PPW)], b_v, sem1)
    cp_m = pltpu.async_copy(mask_hbm.at[:, pl.ds(base * L, PPW * L)], mask_v, sem_in)
    cp_a.wait()
    cp_b.wait()

    def add_row(p, carry):
        for j in range(SL):
            sl = pl.ds(j * L, L)
            a_v[p, sl] = a_v[p, sl] + b_v[p, sl]
        return carry

    lax.fori_loop(0, PPW, add_row, 0)
    cp_m.wait()

    obufs = (ob0_v, ob1_v)
    sems = (sem0, sem1)

    def scale_rows(m, ob):
        def row(p, c):
            mval = mask_v[m, pl.ds(p * L, L)]
            for j in range(SL):
                sl = pl.ds(j * L, L)
                ob[p, sl] = a_v[p, sl] * mval
            return c

        lax.fori_loop(0, PPW, row, 0)

    def out_dma(m, ob, sem):
        return pltpu.async_copy(ob, out_hbm.at[m, pl.ds(base, PPW)], sem)

    scale_rows(0, ob0_v)
    out_dma(0, ob0_v, sem0)
    scale_rows(1, ob1_v)
    out_dma(1, ob1_v, sem1)

    def pair(i, carry):
        m0 = 2 + 2 * i
        for k in range(2):
            ob, sem = obufs[k], sems[k]
            pltpu.make_async_copy(ob, out_hbm.at[m0 + k, pl.ds(base, PPW)], sem).wait()
            scale_rows(m0 + k, ob)
            out_dma(m0 + k, ob, sem)
        return carry

    lax.fori_loop(0, (M - 2) // 2, pair, 0)

    pltpu.make_async_copy(ob0_v, out_hbm.at[0, pl.ds(base, PPW)], sem0).wait()
    pltpu.make_async_copy(ob1_v, out_hbm.at[1, pl.ds(base, PPW)], sem1).wait()


def kernel(image_features, pos_table, masks):
    maskx = jnp.repeat(masks.astype(jnp.float32), L, axis=1)
    return _sc_kernel(image_features, pos_table, maskx)
